# baseline (speedup 1.0000x reference)
import functools

import jax
import jax.numpy as jnp
from jax.experimental import pallas as pl
from jax.experimental.pallas import tpu as pltpu
from jax.experimental.pallas import tpu_sc as plsc

EPS = 1e-12
_GATHER_WINDOW = 64
_NC = 2
_NS = 16
_NCHUNK = 2


def _sc_gather(weight, flat_ids, tt_ids, tt_emb):
    n = flat_ids.shape[0]
    h = weight.shape[1]
    nw = _NC * _NS
    b_per_w = n // nw
    ch = min(_GATHER_WINDOW, b_per_w)
    nchunks = b_per_w // ch
    nhc = h // 16
    mesh = plsc.VectorSubcoreMesh(core_axis_name="core", subcore_axis_name="subcore")

    @functools.partial(
        pl.kernel,
        out_type=jax.ShapeDtypeStruct((n, h), weight.dtype),
        mesh=mesh,
        scratch_types=[
            pltpu.VMEM((b_per_w,), jnp.int32),
            pltpu.VMEM((b_per_w + 16,), jnp.int32),
            pltpu.VMEM((2, h), jnp.float32),
            pltpu.VMEM((2, ch, h), jnp.float32),
            pltpu.SemaphoreType.DMA,
            pltpu.SemaphoreType.DMA,
            pltpu.SemaphoreType.DMA,
            pltpu.SemaphoreType.DMA,
        ],
    )
    def k(table, idx_hbm, tt_hbm, tte_hbm, o_hbm,
          idx_v, ttid_v, tte_v, buf, gs0, gs1, os0, os1):
        gsems = (gs0, gs1)
        osems = (os0, os1)
        c = jax.lax.axis_index("core")
        s = jax.lax.axis_index("subcore")
        wid = s * _NC + c
        base = wid * b_per_w
        pltpu.sync_copy(idx_hbm.at[pl.ds(base, b_per_w)], idx_v)
        pltpu.sync_copy(tt_hbm.at[pl.ds(base, b_per_w)],
                        ttid_v.at[pl.ds(0, b_per_w)])
        pltpu.sync_copy(tte_hbm, tte_v)

        def gstart(j):
            bi = j % 2
            return pltpu.async_copy(
                table.at[idx_v.at[pl.ds(j * ch, ch)]], buf.at[bi], gsems[bi])

        def add_tt(j):
            bi = j % 2

            @pl.loop(0, ch)
            def _(row):
                ttr = ttid_v[pl.ds(j * ch + row, 16)][0]
                for cc in range(nhc):
                    sl = pl.ds(cc * 16, 16)
                    plsc.addupdate(buf.at[bi, row, sl], tte_v[ttr, sl])

        g = [gstart(0)]
        if nchunks > 1:
            g.append(gstart(1))
        o = [None, None]
        for j in range(nchunks):
            bi = j % 2
            g[bi].wait()
            add_tt(j)
            o[bi] = pltpu.async_copy(
                buf.at[bi], o_hbm.at[pl.ds(base + j * ch, ch)], osems[bi])
            if j + 2 < nchunks:
                o[bi].wait()
                g[bi] = gstart(j + 2)
        for j in range(max(0, nchunks - 2), nchunks):
            o[j % 2].wait()

    return k(weight, flat_ids, tt_ids, tt_emb)


def _tc_add_ln(wsum, pos_emb, gamma, beta, seq, out_rows, row0, prev):
    cn, h = wsum.shape
    r = 1024
    pb = seq // r
    bsz = cn // seq
    blk0 = row0 // r

    def body(w_ref, p_ref, g_ref, b_ref, *rest):
        o_ref = rest[-1]
        x = w_ref[...] + p_ref[...]
        xb = x.astype(jnp.bfloat16)
        ones = jnp.ones((h, 128), jnp.bfloat16)
        dn = (((1,), (0,)), ((), ()))
        s1 = jax.lax.dot_general(xb, ones, dn,
                                 preferred_element_type=jnp.float32)
        s2 = jax.lax.dot_general(xb * xb, ones, dn,
                                 preferred_element_type=jnp.float32)
        mean = s1[:, :1] * (1.0 / h)
        ex2 = s2[:, :1] * (1.0 / h)
        rstd = jax.lax.rsqrt(ex2 - mean * mean + EPS)
        o_ref[...] = (x - mean) * rstd * g_ref[...] + b_ref[...]

    in_specs = [
        pl.BlockSpec((r, h), lambda j, b: (b * pb + j, 0)),
        pl.BlockSpec((r, h), lambda j, b: (j, 0)),
        pl.BlockSpec((1, h), lambda j, b: (0, 0)),
        pl.BlockSpec((1, h), lambda j, b: (0, 0)),
    ]
    args = [wsum, pos_emb, gamma.reshape(1, h), beta.reshape(1, h)]
    aliases = {}
    if prev is not None:
        in_specs.append(pl.BlockSpec(memory_space=pl.ANY))
        args.append(prev)
        aliases = {4: 0}
    return pl.pallas_call(
        body,
        grid=(pb, bsz),
        in_specs=in_specs,
        out_specs=pl.BlockSpec((r, h), lambda j, b: (blk0 + b * pb + j, 0)),
        out_shape=jax.ShapeDtypeStruct((out_rows, h), jnp.float32),
        input_output_aliases=aliases,
    )(*args)


def kernel(input_ids, token_type_ids, weight, token_type_embeddings,
           position_embeddings, ln_gamma, ln_beta):
    b, s = input_ids.shape
    h = weight.shape[1]
    n = b * s
    flat_ids = input_ids.reshape(n).astype(jnp.int32)
    flat_tt = token_type_ids.reshape(n).astype(jnp.int32)
    cn = n // _NCHUNK
    wsums = [
        _sc_gather(weight,
                   jax.lax.slice(flat_ids, (ci * cn,), ((ci + 1) * cn,)),
                   jax.lax.slice(flat_tt, (ci * cn,), ((ci + 1) * cn,)),
                   token_type_embeddings)
        for ci in range(_NCHUNK)
    ]
    out = None
    for ci in range(_NCHUNK):
        out = _tc_add_ln(
            wsums[ci], position_embeddings, ln_gamma, ln_beta, s,
            n, ci * cn, out)
    return out.reshape(b, s, h)


# speedup vs baseline: 1.6405x; 1.6405x over previous
import functools

import jax
import jax.numpy as jnp
from jax.experimental import pallas as pl
from jax.experimental.pallas import tpu as pltpu
from jax.experimental.pallas import tpu_sc as plsc

EPS = 1e-12
_GATHER_WINDOW = 64
_NC = 2
_NS = 16
_NCHUNK = 2
_R = 512


def _sc_gather(weight, flat_ids):
    n = flat_ids.shape[0]
    h = weight.shape[1]
    nw = _NC * _NS
    b_per_w = n // nw
    ch = min(_GATHER_WINDOW, b_per_w)
    nchunks = b_per_w // ch
    mesh = plsc.VectorSubcoreMesh(core_axis_name="core", subcore_axis_name="subcore")

    @functools.partial(
        pl.kernel,
        out_type=jax.ShapeDtypeStruct((n, h), weight.dtype),
        mesh=mesh,
        scratch_types=[
            pltpu.VMEM((b_per_w,), jnp.int32),
            pltpu.VMEM((2, ch, h), jnp.float32),
            pltpu.SemaphoreType.DMA,
            pltpu.SemaphoreType.DMA,
            pltpu.SemaphoreType.DMA,
            pltpu.SemaphoreType.DMA,
        ],
    )
    def k(table, idx_hbm, o_hbm, idx_v, buf, gs0, gs1, os0, os1):
        gsems = (gs0, gs1)
        osems = (os0, os1)
        c = jax.lax.axis_index("core")
        s = jax.lax.axis_index("subcore")
        wid = s * _NC + c
        base = wid * b_per_w
        pltpu.sync_copy(idx_hbm.at[pl.ds(base, b_per_w)], idx_v)

        def gstart(j):
            bi = j % 2
            return pltpu.async_copy(
                table.at[idx_v.at[pl.ds(j * ch, ch)]], buf.at[bi], gsems[bi])

        g = [gstart(0)]
        if nchunks > 1:
            g.append(gstart(1))
        o = [None, None]
        for j in range(nchunks):
            bi = j % 2
            g[bi].wait()
            o[bi] = pltpu.async_copy(
                buf.at[bi], o_hbm.at[pl.ds(base + j * ch, ch)], osems[bi])
            if j + 2 < nchunks:
                o[bi].wait()
                g[bi] = gstart(j + 2)
        for j in range(max(0, nchunks - 2), nchunks):
            o[j % 2].wait()

    return k(weight, flat_ids)


def _tc_add_ln(wsum, pos_emb, tt_f, tt_emb, gamma, beta, seq,
               out_rows, row0, prev):
    cn, h = wsum.shape
    r = _R
    pb = seq // r
    bsz = cn // seq
    blk0 = row0 // r

    def body(w_ref, p_ref, t_ref, te_ref, g_ref, b_ref, *rest):
        o_ref = rest[-1]
        x = w_ref[...]
        tt = t_ref[...]
        te = te_ref[...]
        t0 = te[0:1, :]
        t1 = te[1:2, :]
        x = x + p_ref[...] + t0 + tt * (t1 - t0)
        xb = x.astype(jnp.bfloat16)
        ones = jnp.ones((h, 128), jnp.bfloat16)
        dn = (((1,), (0,)), ((), ()))
        s1 = jax.lax.dot_general(xb, ones, dn,
                                 preferred_element_type=jnp.float32)
        s2 = jax.lax.dot_general(xb * xb, ones, dn,
                                 preferred_element_type=jnp.float32)
        mean = s1[:, :1] * (1.0 / h)
        ex2 = s2[:, :1] * (1.0 / h)
        rstd = jax.lax.rsqrt(ex2 - mean * mean + EPS)
        o_ref[...] = (x - mean) * rstd * g_ref[...] + b_ref[...]

    in_specs = [
        pl.BlockSpec((r, h), lambda j, b: (b * pb + j, 0)),
        pl.BlockSpec((r, h), lambda j, b: (j, 0)),
        pl.BlockSpec((r, 1), lambda j, b: (blk0 + b * pb + j, 0)),
        pl.BlockSpec((2, h), lambda j, b: (0, 0)),
        pl.BlockSpec((1, h), lambda j, b: (0, 0)),
        pl.BlockSpec((1, h), lambda j, b: (0, 0)),
    ]
    args = [wsum, pos_emb, tt_f, tt_emb, gamma.reshape(1, h),
            beta.reshape(1, h)]
    aliases = {}
    if prev is not None:
        in_specs.append(pl.BlockSpec(memory_space=pl.ANY))
        args.append(prev)
        aliases = {6: 0}
    return pl.pallas_call(
        body,
        grid=(pb, bsz),
        in_specs=in_specs,
        out_specs=pl.BlockSpec((r, h), lambda j, b: (blk0 + b * pb + j, 0)),
        out_shape=jax.ShapeDtypeStruct((out_rows, h), jnp.float32),
        input_output_aliases=aliases,
    )(*args)


def kernel(input_ids, token_type_ids, weight, token_type_embeddings,
           position_embeddings, ln_gamma, ln_beta):
    b, s = input_ids.shape
    h = weight.shape[1]
    n = b * s
    flat_ids = input_ids.reshape(n).astype(jnp.int32)
    tt_f = token_type_ids.reshape(n, 1).astype(jnp.float32)
    cn = n // _NCHUNK
    wsums = [
        _sc_gather(weight, jax.lax.slice(flat_ids, (ci * cn,), ((ci + 1) * cn,)))
        for ci in range(_NCHUNK)
    ]
    out = None
    for ci in range(_NCHUNK):
        out = _tc_add_ln(
            wsums[ci], position_embeddings, tt_f,
            token_type_embeddings, ln_gamma, ln_beta, s,
            n, ci * cn, out)
    return out.reshape(b, s, h)


# speedup vs baseline: 1.7358x; 1.0581x over previous
import functools

import jax
import jax.numpy as jnp
from jax.experimental import pallas as pl
from jax.experimental.pallas import tpu as pltpu
from jax.experimental.pallas import tpu_sc as plsc

EPS = 1e-12
_GATHER_WINDOW = 64
_NC = 2
_NS = 16
_NCHUNK = 2
_R = 1024


def _sc_gather(weight, flat_ids):
    n = flat_ids.shape[0]
    h = weight.shape[1]
    nw = _NC * _NS
    b_per_w = n // nw
    ch = min(_GATHER_WINDOW, b_per_w)
    nchunks = b_per_w // ch
    mesh = plsc.VectorSubcoreMesh(core_axis_name="core", subcore_axis_name="subcore")

    @functools.partial(
        pl.kernel,
        out_type=jax.ShapeDtypeStruct((n, h), weight.dtype),
        mesh=mesh,
        scratch_types=[
            pltpu.VMEM((b_per_w,), jnp.int32),
            pltpu.VMEM((2, ch, h), jnp.float32),
            pltpu.SemaphoreType.DMA,
            pltpu.SemaphoreType.DMA,
            pltpu.SemaphoreType.DMA,
            pltpu.SemaphoreType.DMA,
        ],
    )
    def k(table, idx_hbm, o_hbm, idx_v, buf, gs0, gs1, os0, os1):
        gsems = (gs0, gs1)
        osems = (os0, os1)
        c = jax.lax.axis_index("core")
        s = jax.lax.axis_index("subcore")
        wid = s * _NC + c
        base = wid * b_per_w
        pltpu.sync_copy(idx_hbm.at[pl.ds(base, b_per_w)], idx_v)

        def gstart(j):
            bi = j % 2
            return pltpu.async_copy(
                table.at[idx_v.at[pl.ds(j * ch, ch)]], buf.at[bi], gsems[bi])

        g = [gstart(0)]
        if nchunks > 1:
            g.append(gstart(1))
        o = [None, None]
        for j in range(nchunks):
            bi = j % 2
            g[bi].wait()
            o[bi] = pltpu.async_copy(
                buf.at[bi], o_hbm.at[pl.ds(base + j * ch, ch)], osems[bi])
            if j + 2 < nchunks:
                o[bi].wait()
                g[bi] = gstart(j + 2)
        for j in range(max(0, nchunks - 2), nchunks):
            o[j % 2].wait()

    return k(weight, flat_ids)


def _tc_add_ln(wsum, pos_emb, tt_f, tt_emb, gamma, beta, seq,
               out_rows, row0, prev):
    cn, h = wsum.shape
    r = _R
    pb = seq // r
    bsz = cn // seq
    blk0 = row0 // r

    def body(w_ref, p_ref, t_ref, te_ref, g_ref, b_ref, *rest):
        o_ref = rest[-1]
        x = w_ref[...]
        tt = t_ref[...].astype(jnp.float32)
        te = te_ref[...]
        t0 = te[0:1, :]
        t1 = te[1:2, :]
        x = x + p_ref[...] + t0 + tt * (t1 - t0)
        xb = x.astype(jnp.bfloat16)
        ones = jnp.ones((h, 128), jnp.bfloat16)
        dn = (((1,), (0,)), ((), ()))
        s1 = jax.lax.dot_general(xb, ones, dn,
                                 preferred_element_type=jnp.float32)
        s2 = jax.lax.dot_general(xb * xb, ones, dn,
                                 preferred_element_type=jnp.float32)
        mean = s1[:, :1] * (1.0 / h)
        ex2 = s2[:, :1] * (1.0 / h)
        rstd = jax.lax.rsqrt(ex2 - mean * mean + EPS)
        o_ref[...] = (x - mean) * rstd * g_ref[...] + b_ref[...]

    in_specs = [
        pl.BlockSpec((r, h), lambda j, b: (b * pb + j, 0)),
        pl.BlockSpec((r, h), lambda j, b: (j, 0)),
        pl.BlockSpec((r, 1), lambda j, b: (blk0 + b * pb + j, 0)),
        pl.BlockSpec((2, h), lambda j, b: (0, 0)),
        pl.BlockSpec((1, h), lambda j, b: (0, 0)),
        pl.BlockSpec((1, h), lambda j, b: (0, 0)),
    ]
    args = [wsum, pos_emb, tt_f, tt_emb, gamma.reshape(1, h),
            beta.reshape(1, h)]
    aliases = {}
    if prev is not None:
        in_specs.append(pl.BlockSpec(memory_space=pl.ANY))
        args.append(prev)
        aliases = {6: 0}
    return pl.pallas_call(
        body,
        grid=(pb, bsz),
        in_specs=in_specs,
        out_specs=pl.BlockSpec((r, h), lambda j, b: (blk0 + b * pb + j, 0)),
        out_shape=jax.ShapeDtypeStruct((out_rows, h), jnp.float32),
        input_output_aliases=aliases,
        compiler_params=pltpu.CompilerParams(
            dimension_semantics=("parallel", "parallel")),
    )(*args)


def kernel(input_ids, token_type_ids, weight, token_type_embeddings,
           position_embeddings, ln_gamma, ln_beta):
    b, s = input_ids.shape
    h = weight.shape[1]
    n = b * s
    flat_ids = input_ids.reshape(n).astype(jnp.int32)
    tt_f = token_type_ids.reshape(n, 1).astype(jnp.bfloat16)
    cn = n // _NCHUNK
    wsums = [
        _sc_gather(weight, jax.lax.slice(flat_ids, (ci * cn,), ((ci + 1) * cn,)))
        for ci in range(_NCHUNK)
    ]
    out = None
    for ci in range(_NCHUNK):
        out = _tc_add_ln(
            wsums[ci], position_embeddings, tt_f,
            token_type_embeddings, ln_gamma, ln_beta, s,
            n, ci * cn, out)
    return out.reshape(b, s, h)
